# speedup vs baseline: 1.7756x; 1.7756x over previous
import jax
import jax.numpy as jnp
from jax.experimental import pallas as pl

N = 10000
D = 128
_MB = 400


def _mm_body(x_ref, w_ref, o_ref):
    o_ref[...] = jnp.dot(x_ref[...], w_ref[...], preferred_element_type=jnp.float32)


def _mm(x, w):
    return pl.pallas_call(
        _mm_body,
        grid=(N // _MB,),
        in_specs=[
            pl.BlockSpec((_MB, D), lambda i: (i, 0)),
            pl.BlockSpec((D, D), lambda i: (0, 0)),
        ],
        out_specs=pl.BlockSpec((_MB, D), lambda i: (i, 0)),
        out_shape=jax.ShapeDtypeStruct((N, D), jnp.float32),
    )(x, w)


def _gat_layer(x, W, a_l, a_r, src, dst):
    hp = _mm(x, W)
    el = hp @ a_l
    er = hp @ a_r
    e = jax.nn.leaky_relu(el[src] + er[dst], negative_slope=0.2)
    w = jnp.exp(e)
    denom = jax.ops.segment_sum(w, dst, num_segments=N)
    num = jax.ops.segment_sum(w[:, None] * hp[src], dst, num_segments=N)
    return jax.nn.elu(num / (denom[:, None] + 1e-9))


def kernel(h, edge_index, W_in, W1, a_l1, a_r1, W2, a_l2, a_r2, W_out):
    src = edge_index[0]
    dst = edge_index[1]
    x = _mm(h, W_in)
    x = _gat_layer(x, W1, a_l1, a_r1, src, dst)
    x = _gat_layer(x, W2, a_l2, a_r2, src, dst)
    return _mm(x, W_out)


# speedup vs baseline: 22.8195x; 12.8519x over previous
import jax
import jax.numpy as jnp
from jax import lax
from jax.experimental import pallas as pl
from jax.experimental.pallas import tpu as pltpu
from jax.experimental.pallas import tpu_sc as plsc

N = 10000
D = 128
E = 320000

NS = 16
EPT = E // NS
CH = 80
NCHUNK = EPT // CH
ROWS_SUB = 624
TAIL = N - NS * ROWS_SUB

_MB = 400


def _mm_body(x_ref, w_ref, o_ref):
    o_ref[...] = jnp.dot(x_ref[...], w_ref[...], preferred_element_type=jnp.float32)


def _mm(x, w):
    return pl.pallas_call(
        _mm_body,
        grid=(N // _MB,),
        in_specs=[
            pl.BlockSpec((_MB, D), lambda i: (i, 0)),
            pl.BlockSpec((D, D), lambda i: (0, 0)),
        ],
        out_specs=pl.BlockSpec((_MB, D), lambda i: (i, 0)),
        out_shape=jax.ShapeDtypeStruct((N, D), jnp.float32),
    )(x, w)


def _proj_body(x_ref, w_ref, a_ref, hp_ref, lr_ref):
    hp = jnp.dot(x_ref[...], w_ref[...], preferred_element_type=jnp.float32)
    hp_ref[...] = hp
    lr_ref[...] = jnp.dot(hp, a_ref[...], preferred_element_type=jnp.float32)


def _proj(x, w, a2):
    return pl.pallas_call(
        _proj_body,
        grid=(N // _MB,),
        in_specs=[
            pl.BlockSpec((_MB, D), lambda i: (i, 0)),
            pl.BlockSpec((D, D), lambda i: (0, 0)),
            pl.BlockSpec((D, 8), lambda i: (0, 0)),
        ],
        out_specs=[
            pl.BlockSpec((_MB, D), lambda i: (i, 0)),
            pl.BlockSpec((_MB, 8), lambda i: (i, 0)),
        ],
        out_shape=[
            jax.ShapeDtypeStruct((N, D), jnp.float32),
            jax.ShapeDtypeStruct((N, 8), jnp.float32),
        ],
    )(x, w, a2)


def _combine_proj_body(n_ref, d_ref, w_ref, a_ref, hp_ref, lr_ref):
    den = d_ref[0, 0, :] + 1e-9
    x = n_ref[...] / den[:, None]
    x = jnp.where(x > 0, x, jnp.exp(x) - 1.0)
    hp = jnp.dot(x, w_ref[...], preferred_element_type=jnp.float32)
    hp_ref[...] = hp
    lr_ref[...] = jnp.dot(hp, a_ref[...], preferred_element_type=jnp.float32)


def _combine_proj(num, den, w, a2):
    den = den.reshape(N // _MB, 1, _MB)
    return pl.pallas_call(
        _combine_proj_body,
        grid=(N // _MB,),
        in_specs=[
            pl.BlockSpec((_MB, D), lambda i: (i, 0)),
            pl.BlockSpec((1, 1, _MB), lambda i: (i, 0, 0)),
            pl.BlockSpec((D, D), lambda i: (0, 0)),
            pl.BlockSpec((D, 8), lambda i: (0, 0)),
        ],
        out_specs=[
            pl.BlockSpec((_MB, D), lambda i: (i, 0)),
            pl.BlockSpec((_MB, 8), lambda i: (i, 0)),
        ],
        out_shape=[
            jax.ShapeDtypeStruct((N, D), jnp.float32),
            jax.ShapeDtypeStruct((N, 8), jnp.float32),
        ],
    )(num, den, w, a2)


def _combine_out_body(n_ref, d_ref, w_ref, o_ref):
    den = d_ref[0, 0, :] + 1e-9
    x = n_ref[...] / den[:, None]
    x = jnp.where(x > 0, x, jnp.exp(x) - 1.0)
    o_ref[...] = jnp.dot(x, w_ref[...], preferred_element_type=jnp.float32)


def _combine_out(num, den, w):
    den = den.reshape(N // _MB, 1, _MB)
    return pl.pallas_call(
        _combine_out_body,
        grid=(N // _MB,),
        in_specs=[
            pl.BlockSpec((_MB, D), lambda i: (i, 0)),
            pl.BlockSpec((1, 1, _MB), lambda i: (i, 0, 0)),
            pl.BlockSpec((D, D), lambda i: (0, 0)),
        ],
        out_specs=pl.BlockSpec((_MB, D), lambda i: (i, 0)),
        out_shape=jax.ShapeDtypeStruct((N, D), jnp.float32),
    )(num, den, w)


def _edge_body(hp, el, er, srcm, dstm, znd, zn, num_o, den_o,
               el_v, er_v, srcc, dstc, wc, rows_v, num_s, den_s,
               isem, gsem, ssem):
    s = lax.axis_index("s")

    pltpu.sync_copy(znd.at[pl.ds(s * ROWS_SUB, ROWS_SUB)],
                    num_s.at[pl.ds(s * ROWS_SUB, ROWS_SUB)])

    @pl.when(s == NS - 1)
    def _():
        pltpu.sync_copy(znd.at[pl.ds(NS * ROWS_SUB, TAIL)],
                        num_s.at[pl.ds(NS * ROWS_SUB, TAIL)])

    @pl.when(s == 0)
    def _():
        pltpu.sync_copy(zn, den_s)

    pltpu.sync_copy(el, el_v)
    pltpu.sync_copy(er, er_v)

    plsc.subcore_barrier()

    def idx_fetch(ci, b):
        pltpu.async_copy(srcm.at[s, ci], srcc.at[b], isem.at[b])
        pltpu.async_copy(dstm.at[s, ci], dstc.at[b], isem.at[b])

    def idx_wait(ci, b):
        pltpu.make_async_copy(srcm.at[s, ci], srcc.at[b], isem.at[b]).wait()
        pltpu.make_async_copy(dstm.at[s, ci], dstc.at[b], isem.at[b]).wait()

    def compute_w(b):
        for j in range(CH // 16):
            sl = pl.ds(j * 16, 16)
            s16 = srcc[b, sl]
            d16 = dstc[b, sl]
            x = plsc.load_gather(el_v, [s16]) + plsc.load_gather(er_v, [d16])
            e = jnp.maximum(x, 0.2 * x)
            wc[b, sl] = jnp.exp(e)

    idx_fetch(0, 0)
    idx_wait(0, 0)
    compute_w(0)
    pltpu.async_copy(hp.at[srcc.at[0]], rows_v.at[0], gsem.at[0])
    idx_fetch(1, 1)

    def _iter(ci, carry):
        b2 = lax.rem(ci, 2)
        nb2 = 1 - b2
        b4 = lax.rem(ci, 4)

        @pl.when(ci >= 1)
        def _():
            pltpu.make_async_copy(
                rows_v.at[nb2], num_s.at[dstc.at[lax.rem(ci + 3, 4)]],
                ssem.at[nb2]).wait()

        @pl.when(ci + 1 < NCHUNK)
        def _():
            nb4 = lax.rem(ci + 1, 4)
            idx_wait(ci + 1, nb4)
            compute_w(nb4)
            pltpu.async_copy(hp.at[srcc.at[nb4]], rows_v.at[nb2], gsem.at[nb2])

        @pl.when(ci + 2 < NCHUNK)
        def _():
            idx_fetch(ci + 2, lax.rem(ci + 2, 4))

        pltpu.make_async_copy(hp.at[srcc.at[b4]], rows_v.at[b2],
                              gsem.at[b2]).wait()

        b16 = jnp.full((16,), b4, jnp.int32)

        def _scale(i, carry2):
            w16 = plsc.load_gather(wc, [b16, jnp.full((16,), i, jnp.int32)])
            for j in range(D // 16):
                sl = pl.ds(j * 16, 16)
                rows_v[b2, i, sl] = rows_v[b2, i, sl] * w16
            return carry2

        lax.fori_loop(0, CH, _scale, 0)

        pltpu.sync_copy(wc.at[b4], den_s.at[dstc.at[b4]], add=True)
        pltpu.async_copy(rows_v.at[b2], num_s.at[dstc.at[b4]], ssem.at[b2],
                         add=True)
        return carry

    lax.fori_loop(0, NCHUNK, _iter, 0)

    pltpu.make_async_copy(
        rows_v.at[lax.rem(NCHUNK - 1, 2)],
        num_s.at[dstc.at[lax.rem(NCHUNK - 1, 4)]],
        ssem.at[lax.rem(NCHUNK - 1, 2)]).wait()

    plsc.subcore_barrier()

    pltpu.sync_copy(num_s.at[pl.ds(s * ROWS_SUB, ROWS_SUB)],
                    num_o.at[pl.ds(s * ROWS_SUB, ROWS_SUB)])

    @pl.when(s == NS - 1)
    def _():
        pltpu.sync_copy(num_s.at[pl.ds(NS * ROWS_SUB, TAIL)],
                        num_o.at[pl.ds(NS * ROWS_SUB, TAIL)])

    @pl.when(s == 0)
    def _():
        pltpu.sync_copy(den_s, den_o)


_edge = pl.kernel(
    _edge_body,
    out_type=[
        jax.ShapeDtypeStruct((N, D), jnp.float32),
        jax.ShapeDtypeStruct((N,), jnp.float32),
    ],
    mesh=plsc.VectorSubcoreMesh(core_axis_name="c", subcore_axis_name="s",
                                num_cores=1, num_subcores=NS),
    compiler_params=pltpu.CompilerParams(needs_layout_passes=False),
    scratch_types=[
        pltpu.VMEM((N,), jnp.float32),
        pltpu.VMEM((N,), jnp.float32),
        pltpu.VMEM((4, CH), jnp.int32),
        pltpu.VMEM((4, CH), jnp.int32),
        pltpu.VMEM((4, CH), jnp.float32),
        pltpu.VMEM((2, CH, D), jnp.float32),
        pltpu.VMEM_SHARED((N, D), jnp.float32),
        pltpu.VMEM_SHARED((N,), jnp.float32),
        pltpu.SemaphoreType.DMA((4,)),
        pltpu.SemaphoreType.DMA((2,)),
        pltpu.SemaphoreType.DMA((2,)),
    ],
)


def _pack_a(a_l, a_r):
    a2 = jnp.zeros((D, 8), jnp.float32)
    return a2.at[:, 0].set(a_l).at[:, 1].set(a_r)


def kernel(h, edge_index, W_in, W1, a_l1, a_r1, W2, a_l2, a_r2, W_out):
    srcm = edge_index[0].reshape(NS, NCHUNK, CH)
    dstm = edge_index[1].reshape(NS, NCHUNK, CH)
    znd = jnp.zeros((N, D), jnp.float32)
    zn = jnp.zeros((N,), jnp.float32)

    x0 = _mm(h, W_in)

    hp1, lr1 = _proj(x0, W1, _pack_a(a_l1, a_r1))
    num1, den1 = _edge(hp1, lr1[:, 0], lr1[:, 1], srcm, dstm, znd, zn)

    hp2, lr2 = _combine_proj(num1, den1, W2, _pack_a(a_l2, a_r2))
    num2, den2 = _edge(hp2, lr2[:, 0], lr2[:, 1], srcm, dstm, znd, zn)

    return _combine_out(num2, den2, W_out)


# speedup vs baseline: 29.4698x; 1.2914x over previous
import jax
import jax.numpy as jnp
from jax import lax
from jax.experimental import pallas as pl
from jax.experimental.pallas import tpu as pltpu
from jax.experimental.pallas import tpu_sc as plsc

N = 10000
D = 128
E = 320000

NS = 16
EPT = E // NS
CH = 80
NCHUNK = EPT // CH
ROWS_SUB = 624
TAIL = N - NS * ROWS_SUB

_MB = 400


def _mm_body(x_ref, w_ref, o_ref):
    o_ref[...] = jnp.dot(x_ref[...], w_ref[...], preferred_element_type=jnp.float32)


def _mm(x, w):
    return pl.pallas_call(
        _mm_body,
        grid=(N // _MB,),
        in_specs=[
            pl.BlockSpec((_MB, D), lambda i: (i, 0)),
            pl.BlockSpec((D, D), lambda i: (0, 0)),
        ],
        out_specs=pl.BlockSpec((_MB, D), lambda i: (i, 0)),
        out_shape=jax.ShapeDtypeStruct((N, D), jnp.float32),
    )(x, w)


def _proj_body(x_ref, w_ref, a_ref, hp_ref, lr_ref):
    hp = jnp.dot(x_ref[...], w_ref[...], preferred_element_type=jnp.float32)
    hp_ref[...] = hp
    lr_ref[...] = jnp.dot(hp, a_ref[...], preferred_element_type=jnp.float32)


def _proj(x, w, a2):
    return pl.pallas_call(
        _proj_body,
        grid=(N // _MB,),
        in_specs=[
            pl.BlockSpec((_MB, D), lambda i: (i, 0)),
            pl.BlockSpec((D, D), lambda i: (0, 0)),
            pl.BlockSpec((D, 8), lambda i: (0, 0)),
        ],
        out_specs=[
            pl.BlockSpec((_MB, D), lambda i: (i, 0)),
            pl.BlockSpec((_MB, 8), lambda i: (i, 0)),
        ],
        out_shape=[
            jax.ShapeDtypeStruct((N, D), jnp.float32),
            jax.ShapeDtypeStruct((N, 8), jnp.float32),
        ],
    )(x, w, a2)


def _combine_proj_body(n_ref, d_ref, w_ref, a_ref, hp_ref, lr_ref):
    den = d_ref[0, 0, :] + 1e-9
    x = n_ref[...] / den[:, None]
    x = jnp.where(x > 0, x, jnp.exp(x) - 1.0)
    hp = jnp.dot(x, w_ref[...], preferred_element_type=jnp.float32)
    hp_ref[...] = hp
    lr_ref[...] = jnp.dot(hp, a_ref[...], preferred_element_type=jnp.float32)


def _combine_proj(num, den, w, a2):
    den = den.reshape(N // _MB, 1, _MB)
    return pl.pallas_call(
        _combine_proj_body,
        grid=(N // _MB,),
        in_specs=[
            pl.BlockSpec((_MB, D), lambda i: (i, 0)),
            pl.BlockSpec((1, 1, _MB), lambda i: (i, 0, 0)),
            pl.BlockSpec((D, D), lambda i: (0, 0)),
            pl.BlockSpec((D, 8), lambda i: (0, 0)),
        ],
        out_specs=[
            pl.BlockSpec((_MB, D), lambda i: (i, 0)),
            pl.BlockSpec((_MB, 8), lambda i: (i, 0)),
        ],
        out_shape=[
            jax.ShapeDtypeStruct((N, D), jnp.float32),
            jax.ShapeDtypeStruct((N, 8), jnp.float32),
        ],
    )(num, den, w, a2)


def _combine_out_body(n_ref, d_ref, w_ref, o_ref):
    den = d_ref[0, 0, :] + 1e-9
    x = n_ref[...] / den[:, None]
    x = jnp.where(x > 0, x, jnp.exp(x) - 1.0)
    o_ref[...] = jnp.dot(x, w_ref[...], preferred_element_type=jnp.float32)


def _combine_out(num, den, w):
    den = den.reshape(N // _MB, 1, _MB)
    return pl.pallas_call(
        _combine_out_body,
        grid=(N // _MB,),
        in_specs=[
            pl.BlockSpec((_MB, D), lambda i: (i, 0)),
            pl.BlockSpec((1, 1, _MB), lambda i: (i, 0, 0)),
            pl.BlockSpec((D, D), lambda i: (0, 0)),
        ],
        out_specs=pl.BlockSpec((_MB, D), lambda i: (i, 0)),
        out_shape=jax.ShapeDtypeStruct((N, D), jnp.float32),
    )(num, den, w)


def _edge_body(hp, el, er, srcm, dstm, znd, zn, num_o, den_o,
               el_v, er_v, srcc, dstc, wc, rows_v, num_s, den_s,
               isem, gsem, ssem, dsem):
    s = lax.axis_index("s")

    pltpu.sync_copy(znd.at[pl.ds(s * ROWS_SUB, ROWS_SUB)],
                    num_s.at[pl.ds(s * ROWS_SUB, ROWS_SUB)])

    @pl.when(s == NS - 1)
    def _():
        pltpu.sync_copy(znd.at[pl.ds(NS * ROWS_SUB, TAIL)],
                        num_s.at[pl.ds(NS * ROWS_SUB, TAIL)])

    @pl.when(s == 0)
    def _():
        pltpu.sync_copy(zn, den_s)

    pltpu.sync_copy(el, el_v)
    pltpu.sync_copy(er, er_v)

    plsc.subcore_barrier()

    def idx_fetch(ci, b):
        pltpu.async_copy(srcm.at[s, ci], srcc.at[b], isem.at[b])
        pltpu.async_copy(dstm.at[s, ci], dstc.at[b], isem.at[b])

    def idx_wait(ci, b):
        pltpu.make_async_copy(srcm.at[s, ci], srcc.at[b], isem.at[b]).wait()
        pltpu.make_async_copy(dstm.at[s, ci], dstc.at[b], isem.at[b]).wait()

    def compute_w(b):
        for j in range(CH // 16):
            sl = pl.ds(j * 16, 16)
            s16 = srcc[b, sl]
            d16 = dstc[b, sl]
            x = plsc.load_gather(el_v, [s16]) + plsc.load_gather(er_v, [d16])
            e = jnp.maximum(x, 0.2 * x)
            wc[b, sl] = jnp.exp(e)

    idx_fetch(0, 0)
    idx_wait(0, 0)
    compute_w(0)
    pltpu.async_copy(hp.at[srcc.at[0]], rows_v.at[0], gsem.at[0])
    idx_fetch(1, 1)

    def _iter(ci, carry):
        b2 = lax.rem(ci, 2)
        nb2 = 1 - b2
        b4 = lax.rem(ci, 4)

        @pl.when(ci >= 1)
        def _():
            pltpu.make_async_copy(
                rows_v.at[nb2], num_s.at[dstc.at[lax.rem(ci + 3, 4)]],
                ssem.at[nb2]).wait()

        @pl.when(ci + 1 < NCHUNK)
        def _():
            nb4 = lax.rem(ci + 1, 4)
            idx_wait(ci + 1, nb4)
            compute_w(nb4)
            pltpu.async_copy(hp.at[srcc.at[nb4]], rows_v.at[nb2], gsem.at[nb2])

        @pl.when(ci + 2 < NCHUNK)
        def _():
            bf = lax.rem(ci + 2, 4)

            @pl.when(ci >= 2)
            def _():
                pltpu.make_async_copy(wc.at[bf], den_s.at[dstc.at[bf]],
                                      dsem.at[bf]).wait()

            idx_fetch(ci + 2, bf)

        pltpu.make_async_copy(hp.at[srcc.at[b4]], rows_v.at[b2],
                              gsem.at[b2]).wait()

        for g in range(CH // 16):
            w16 = wc[b4, pl.ds(g * 16, 16)]
            for k in range(16):
                w1 = w16[k]
                for j in range(D // 16):
                    sl = pl.ds(j * 16, 16)
                    i = g * 16 + k
                    rows_v[b2, i, sl] = rows_v[b2, i, sl] * w1

        pltpu.async_copy(wc.at[b4], den_s.at[dstc.at[b4]], dsem.at[b4],
                         add=True)
        pltpu.async_copy(rows_v.at[b2], num_s.at[dstc.at[b4]], ssem.at[b2],
                         add=True)
        return carry

    lax.fori_loop(0, NCHUNK, _iter, 0)

    pltpu.make_async_copy(
        rows_v.at[lax.rem(NCHUNK - 1, 2)],
        num_s.at[dstc.at[lax.rem(NCHUNK - 1, 4)]],
        ssem.at[lax.rem(NCHUNK - 1, 2)]).wait()
    for k in range(4):
        b = (NCHUNK - 4 + k) % 4
        pltpu.make_async_copy(wc.at[b], den_s.at[dstc.at[b]],
                              dsem.at[b]).wait()

    plsc.subcore_barrier()

    pltpu.sync_copy(num_s.at[pl.ds(s * ROWS_SUB, ROWS_SUB)],
                    num_o.at[pl.ds(s * ROWS_SUB, ROWS_SUB)])

    @pl.when(s == NS - 1)
    def _():
        pltpu.sync_copy(num_s.at[pl.ds(NS * ROWS_SUB, TAIL)],
                        num_o.at[pl.ds(NS * ROWS_SUB, TAIL)])

    @pl.when(s == 0)
    def _():
        pltpu.sync_copy(den_s, den_o)


_edge = pl.kernel(
    _edge_body,
    out_type=[
        jax.ShapeDtypeStruct((N, D), jnp.float32),
        jax.ShapeDtypeStruct((N,), jnp.float32),
    ],
    mesh=plsc.VectorSubcoreMesh(core_axis_name="c", subcore_axis_name="s",
                                num_cores=1, num_subcores=NS),
    compiler_params=pltpu.CompilerParams(needs_layout_passes=False),
    scratch_types=[
        pltpu.VMEM((N,), jnp.float32),
        pltpu.VMEM((N,), jnp.float32),
        pltpu.VMEM((4, CH), jnp.int32),
        pltpu.VMEM((4, CH), jnp.int32),
        pltpu.VMEM((4, CH), jnp.float32),
        pltpu.VMEM((2, CH, D), jnp.float32),
        pltpu.VMEM_SHARED((N, D), jnp.float32),
        pltpu.VMEM_SHARED((N,), jnp.float32),
        pltpu.SemaphoreType.DMA((4,)),
        pltpu.SemaphoreType.DMA((2,)),
        pltpu.SemaphoreType.DMA((2,)),
        pltpu.SemaphoreType.DMA((4,)),
    ],
)


def _pack_a(a_l, a_r):
    a2 = jnp.zeros((D, 8), jnp.float32)
    return a2.at[:, 0].set(a_l).at[:, 1].set(a_r)


def kernel(h, edge_index, W_in, W1, a_l1, a_r1, W2, a_l2, a_r2, W_out):
    srcm = edge_index[0].reshape(NS, NCHUNK, CH)
    dstm = edge_index[1].reshape(NS, NCHUNK, CH)
    znd = jnp.zeros((N, D), jnp.float32)
    zn = jnp.zeros((N,), jnp.float32)

    x0 = _mm(h, W_in)

    hp1, lr1 = _proj(x0, W1, _pack_a(a_l1, a_r1))
    num1, den1 = _edge(hp1, lr1[:, 0], lr1[:, 1], srcm, dstm, znd, zn)

    hp2, lr2 = _combine_proj(num1, den1, W2, _pack_a(a_l2, a_r2))
    num2, den2 = _edge(hp2, lr2[:, 0], lr2[:, 1], srcm, dstm, znd, zn)

    return _combine_out(num2, den2, W_out)


# speedup vs baseline: 35.3777x; 1.2005x over previous
import jax
import jax.numpy as jnp
from jax import lax
from jax.experimental import pallas as pl
from jax.experimental.pallas import tpu as pltpu
from jax.experimental.pallas import tpu_sc as plsc

N = 10000
D = 128
E = 320000

NS = 16
EPT = E // NS
CH = 40
NCHUNK = EPT // CH
ROWS_SUB = 624
TAIL = N - NS * ROWS_SUB

_MB = 400


def _mm_body(x_ref, w_ref, o_ref):
    o_ref[...] = jnp.dot(x_ref[...], w_ref[...], preferred_element_type=jnp.float32)


def _mm(x, w):
    return pl.pallas_call(
        _mm_body,
        grid=(N // _MB,),
        in_specs=[
            pl.BlockSpec((_MB, D), lambda i: (i, 0)),
            pl.BlockSpec((D, D), lambda i: (0, 0)),
        ],
        out_specs=pl.BlockSpec((_MB, D), lambda i: (i, 0)),
        out_shape=jax.ShapeDtypeStruct((N, D), jnp.float32),
    )(x, w)


def _proj_body(x_ref, w_ref, a_ref, hp_ref, lr_ref):
    hp = jnp.dot(x_ref[...], w_ref[...], preferred_element_type=jnp.float32)
    hp_ref[...] = hp
    lr_ref[...] = jnp.dot(hp, a_ref[...], preferred_element_type=jnp.float32)


def _proj(x, w, a2):
    return pl.pallas_call(
        _proj_body,
        grid=(N // _MB,),
        in_specs=[
            pl.BlockSpec((_MB, D), lambda i: (i, 0)),
            pl.BlockSpec((D, D), lambda i: (0, 0)),
            pl.BlockSpec((D, 8), lambda i: (0, 0)),
        ],
        out_specs=[
            pl.BlockSpec((_MB, D), lambda i: (i, 0)),
            pl.BlockSpec((_MB, 8), lambda i: (i, 0)),
        ],
        out_shape=[
            jax.ShapeDtypeStruct((N, D), jnp.float32),
            jax.ShapeDtypeStruct((N, 8), jnp.float32),
        ],
    )(x, w, a2)


def _combine_proj_body(n_ref, d_ref, w_ref, a_ref, hp_ref, lr_ref):
    den = d_ref[0, 0, :] + 1e-9
    x = n_ref[...] / den[:, None]
    x = jnp.where(x > 0, x, jnp.exp(x) - 1.0)
    hp = jnp.dot(x, w_ref[...], preferred_element_type=jnp.float32)
    hp_ref[...] = hp
    lr_ref[...] = jnp.dot(hp, a_ref[...], preferred_element_type=jnp.float32)


def _combine_proj(num, den, w, a2):
    den = den.reshape(N // _MB, 1, _MB)
    return pl.pallas_call(
        _combine_proj_body,
        grid=(N // _MB,),
        in_specs=[
            pl.BlockSpec((_MB, D), lambda i: (i, 0)),
            pl.BlockSpec((1, 1, _MB), lambda i: (i, 0, 0)),
            pl.BlockSpec((D, D), lambda i: (0, 0)),
            pl.BlockSpec((D, 8), lambda i: (0, 0)),
        ],
        out_specs=[
            pl.BlockSpec((_MB, D), lambda i: (i, 0)),
            pl.BlockSpec((_MB, 8), lambda i: (i, 0)),
        ],
        out_shape=[
            jax.ShapeDtypeStruct((N, D), jnp.float32),
            jax.ShapeDtypeStruct((N, 8), jnp.float32),
        ],
    )(num, den, w, a2)


def _combine_out_body(n_ref, d_ref, w_ref, o_ref):
    den = d_ref[0, 0, :] + 1e-9
    x = n_ref[...] / den[:, None]
    x = jnp.where(x > 0, x, jnp.exp(x) - 1.0)
    o_ref[...] = jnp.dot(x, w_ref[...], preferred_element_type=jnp.float32)


def _combine_out(num, den, w):
    den = den.reshape(N // _MB, 1, _MB)
    return pl.pallas_call(
        _combine_out_body,
        grid=(N // _MB,),
        in_specs=[
            pl.BlockSpec((_MB, D), lambda i: (i, 0)),
            pl.BlockSpec((1, 1, _MB), lambda i: (i, 0, 0)),
            pl.BlockSpec((D, D), lambda i: (0, 0)),
        ],
        out_specs=pl.BlockSpec((_MB, D), lambda i: (i, 0)),
        out_shape=jax.ShapeDtypeStruct((N, D), jnp.float32),
    )(num, den, w)


def _edge_body(hp, el, er, srcm, dstm, znd, zn, num_o, den_o,
               el_v, er_v, srcc, dstc, wc, rows_v, num_s, den_s,
               isem, gsem, ssem, dsem):
    s = lax.axis_index("s")

    pltpu.sync_copy(znd.at[pl.ds(s * ROWS_SUB, ROWS_SUB)],
                    num_s.at[pl.ds(s * ROWS_SUB, ROWS_SUB)])

    @pl.when(s == NS - 1)
    def _():
        pltpu.sync_copy(znd.at[pl.ds(NS * ROWS_SUB, TAIL)],
                        num_s.at[pl.ds(NS * ROWS_SUB, TAIL)])

    @pl.when(s == 0)
    def _():
        pltpu.sync_copy(zn, den_s)

    pltpu.sync_copy(el, el_v)
    pltpu.sync_copy(er, er_v)

    plsc.subcore_barrier()

    def idx_fetch(ci, b):
        pltpu.async_copy(srcm.at[s, ci], srcc.at[b], isem.at[b])
        pltpu.async_copy(dstm.at[s, ci], dstc.at[b], isem.at[b])

    def idx_wait(ci, b):
        pltpu.make_async_copy(srcm.at[s, ci], srcc.at[b], isem.at[b]).wait()
        pltpu.make_async_copy(dstm.at[s, ci], dstc.at[b], isem.at[b]).wait()

    def compute_w(b):
        for j in range(CH // 16):
            sl = pl.ds(j * 16, 16)
            s16 = srcc[b, sl]
            d16 = dstc[b, sl]
            x = plsc.load_gather(el_v, [s16]) + plsc.load_gather(er_v, [d16])
            e = jnp.maximum(x, 0.2 * x)
            wc[b, sl] = jnp.exp(e)

    def num_wait(cj):
        br = lax.rem(cj, 4)
        pltpu.make_async_copy(rows_v.at[br], num_s.at[dstc.at[lax.rem(cj, 8)]],
                              ssem.at[br]).wait()

    def den_wait(cj):
        b8 = lax.rem(cj, 8)
        pltpu.make_async_copy(wc.at[b8], den_s.at[dstc.at[b8]],
                              dsem.at[b8]).wait()

    def prep(ci):
        b8 = lax.rem(ci, 8)
        idx_wait(ci, b8)
        compute_w(b8)
        pltpu.async_copy(hp.at[srcc.at[b8]], rows_v.at[lax.rem(ci, 4)],
                         gsem.at[lax.rem(ci, 4)])

    for ci in range(4):
        idx_fetch(ci, ci)
    prep(0)
    prep(1)

    def _iter(ci, carry):
        br = lax.rem(ci, 4)
        b8 = lax.rem(ci, 8)

        @pl.when(ci >= 2)
        def _():
            num_wait(ci - 2)

        @pl.when(ci + 2 < NCHUNK)
        def _():
            prep(ci + 2)

        @pl.when(ci + 4 < NCHUNK)
        def _():
            @pl.when(ci >= 4)
            def _():
                den_wait(ci - 4)

            idx_fetch(ci + 4, lax.rem(ci + 4, 8))

        pltpu.make_async_copy(hp.at[srcc.at[b8]], rows_v.at[br],
                              gsem.at[br]).wait()

        for g in range(CH // 16):
            w16 = wc[b8, pl.ds(g * 16, 16)]
            for k in range(16):
                w1 = w16[k]
                for j in range(D // 16):
                    sl = pl.ds(j * 16, 16)
                    i = g * 16 + k
                    rows_v[br, i, sl] = rows_v[br, i, sl] * w1

        pltpu.async_copy(wc.at[b8], den_s.at[dstc.at[b8]], dsem.at[b8],
                         add=True)
        pltpu.async_copy(rows_v.at[br], num_s.at[dstc.at[b8]], ssem.at[br],
                         add=True)
        return carry

    lax.fori_loop(0, NCHUNK, _iter, 0)

    num_wait(NCHUNK - 2)
    num_wait(NCHUNK - 1)
    for k in range(8):
        den_wait(NCHUNK - 8 + k)

    plsc.subcore_barrier()

    pltpu.sync_copy(num_s.at[pl.ds(s * ROWS_SUB, ROWS_SUB)],
                    num_o.at[pl.ds(s * ROWS_SUB, ROWS_SUB)])

    @pl.when(s == NS - 1)
    def _():
        pltpu.sync_copy(num_s.at[pl.ds(NS * ROWS_SUB, TAIL)],
                        num_o.at[pl.ds(NS * ROWS_SUB, TAIL)])

    @pl.when(s == 0)
    def _():
        pltpu.sync_copy(den_s, den_o)


_edge = pl.kernel(
    _edge_body,
    out_type=[
        jax.ShapeDtypeStruct((N, D), jnp.float32),
        jax.ShapeDtypeStruct((N,), jnp.float32),
    ],
    mesh=plsc.VectorSubcoreMesh(core_axis_name="c", subcore_axis_name="s",
                                num_cores=1, num_subcores=NS),
    compiler_params=pltpu.CompilerParams(needs_layout_passes=False),
    scratch_types=[
        pltpu.VMEM((N,), jnp.float32),
        pltpu.VMEM((N,), jnp.float32),
        pltpu.VMEM((8, CH), jnp.int32),
        pltpu.VMEM((8, CH), jnp.int32),
        pltpu.VMEM((8, CH), jnp.float32),
        pltpu.VMEM((4, CH, D), jnp.float32),
        pltpu.VMEM_SHARED((N, D), jnp.float32),
        pltpu.VMEM_SHARED((N,), jnp.float32),
        pltpu.SemaphoreType.DMA((8,)),
        pltpu.SemaphoreType.DMA((4,)),
        pltpu.SemaphoreType.DMA((4,)),
        pltpu.SemaphoreType.DMA((8,)),
    ],
)


def _pack_a(a_l, a_r):
    a2 = jnp.zeros((D, 8), jnp.float32)
    return a2.at[:, 0].set(a_l).at[:, 1].set(a_r)


def kernel(h, edge_index, W_in, W1, a_l1, a_r1, W2, a_l2, a_r2, W_out):
    srcm = edge_index[0].reshape(NS, NCHUNK, CH)
    dstm = edge_index[1].reshape(NS, NCHUNK, CH)
    znd = jnp.zeros((N, D), jnp.float32)
    zn = jnp.zeros((N,), jnp.float32)

    x0 = _mm(h, W_in)

    hp1, lr1 = _proj(x0, W1, _pack_a(a_l1, a_r1))
    num1, den1 = _edge(hp1, lr1[:, 0], lr1[:, 1], srcm, dstm, znd, zn)

    hp2, lr2 = _combine_proj(num1, den1, W2, _pack_a(a_l2, a_r2))
    num2, den2 = _edge(hp2, lr2[:, 0], lr2[:, 1], srcm, dstm, znd, zn)

    return _combine_out(num2, den2, W_out)
